# baseline (speedup 1.0000x reference)
import jax
import jax.numpy as jnp
from jax.experimental import pallas as pl
from jax.experimental.pallas import tpu as pltpu

_NBUF = 8
_K = 4
_CHUNK_B = 64


def _dma_pipe(x_ref, o_ref, *scratch):
    bufs = scratch[:_NBUF]
    in_sems = scratch[_NBUF:2 * _NBUF]
    out_sems = scratch[2 * _NBUF:]
    C = x_ref.shape[0] // _CHUNK_B

    def in_copy(i):
        s = i % _NBUF
        return pltpu.make_async_copy(
            x_ref.at[pl.ds(i * _CHUNK_B, _CHUNK_B)], bufs[s], in_sems[s])

    def out_copy(i):
        s = i % _NBUF
        return pltpu.make_async_copy(
            bufs[s], o_ref.at[pl.ds(i * _CHUNK_B, _CHUNK_B)], out_sems[s])

    waited_outs = set()
    for j in range(min(_K, C)):
        in_copy(j).start()
    for i in range(C):
        j = i + _K
        if j < C:
            if j - _NBUF >= 0:
                out_copy(j - _NBUF).wait()
                waited_outs.add(j - _NBUF)
            in_copy(j).start()
        in_copy(i).wait()
        out_copy(i).start()
    for i in range(C):
        if i not in waited_outs:
            out_copy(i).wait()


def kernel(x, hippocampus, neocortex):
    B, S, H = x.shape
    return pl.pallas_call(
        _dma_pipe,
        out_shape=jax.ShapeDtypeStruct(x.shape, x.dtype),
        in_specs=[pl.BlockSpec(memory_space=pl.ANY)],
        out_specs=pl.BlockSpec(memory_space=pl.ANY),
        scratch_shapes=(
            [pltpu.VMEM((_CHUNK_B, S, H), x.dtype) for _ in range(_NBUF)]
            + [pltpu.SemaphoreType.DMA for _ in range(_NBUF)]
            + [pltpu.SemaphoreType.DMA for _ in range(_NBUF)]
        ),
    )(x)
